# baseline (speedup 1.0000x reference)
import functools

import jax
import jax.numpy as jnp
from jax import lax
from jax.experimental import pallas as pl
from jax.experimental.pallas import tpu as pltpu
from jax.experimental.pallas import tpu_sc as plsc

BASE_GSD = 10
EMBED = 256
DIMQ = EMBED // 4
B, H, W, T, BS = 4, 16, 16, 12, 3
HH = 4


def _month_table():
    months = jnp.arange(12, dtype=jnp.float32)
    angles = 2.0 * jnp.pi * months / 12.0
    half = DIMQ // 2
    freq = jnp.arange(1, half + 1, dtype=jnp.float32)
    arg = angles[:, None] * freq[None, :]
    return jnp.concatenate([jnp.sin(arg), jnp.cos(arg)], axis=-1)


def _pos_embed():
    half = DIMQ // 2
    omega = 1.0 / (10000.0 ** (jnp.arange(half, dtype=jnp.float32) / half))
    out = jnp.arange(T, dtype=jnp.float32)[:, None] * omega
    return jnp.concatenate([jnp.sin(out), jnp.cos(out)], axis=-1)


def _month_gather_sc(mtab, months_flat):
    mesh = plsc.VectorSubcoreMesh(core_axis_name="c", subcore_axis_name="s")

    @functools.partial(
        pl.kernel,
        mesh=mesh,
        out_type=jax.ShapeDtypeStruct((B * T, 2 * DIMQ), jnp.float32),
        scratch_types=[
            pltpu.VMEM((B * T,), jnp.int32),
            pltpu.VMEM((B * T, 2 * DIMQ), jnp.float32),
            pltpu.SemaphoreType.DMA,
        ],
    )
    def gather_kernel(mtab_hbm, idx_hbm, out_hbm, idx_v, rows_v, sem):
        wid = lax.axis_index("s") * 2 + lax.axis_index("c")

        @pl.when(wid == 0)
        def _():
            pltpu.sync_copy(idx_hbm, idx_v)
            pltpu.async_copy(mtab_hbm.at[idx_v], rows_v, sem).wait()
            pltpu.sync_copy(rows_v, out_hbm)

    return gather_kernel(mtab, months_flat)


def _embed_add_kernel(tok_ref, me_ref, pe_ref, ch_ref, gsd_ref,
                      out_ref):
    j = pl.program_id(1)
    me = me_ref[0]
    pe = pe_ref[...]
    ch = ch_ref[...]
    gsd = gsd_ref[0, 0]

    add_ts = jnp.concatenate([
        jnp.broadcast_to(ch[None, :, :], (T, BS, DIMQ)),
        jnp.broadcast_to(pe[:, None, :], (T, BS, DIMQ)),
        jnp.broadcast_to(me[:, None, :], (T, BS, DIMQ)),
        jnp.zeros((T, BS, DIMQ), jnp.float32),
    ], axis=-1)

    half2 = DIMQ // 4
    kf = lax.broadcasted_iota(jnp.int32, (1, half2), 1).astype(jnp.float32)
    omega = jnp.exp(kf * (-jnp.log(10000.0) / half2))
    posf = lax.broadcasted_iota(jnp.int32, (W, 1), 0).astype(jnp.float32) * gsd
    arg = posf * omega
    E = jnp.concatenate([jnp.sin(arg), jnp.cos(arg)], axis=-1)
    hrow = (j * HH + lax.broadcasted_iota(jnp.int32, (HH, 1), 0))
    argh = hrow.astype(jnp.float32) * gsd * omega
    Eh = jnp.concatenate([jnp.sin(argh), jnp.cos(argh)], axis=-1)
    S4 = jnp.concatenate([
        jnp.broadcast_to(Eh[:, None, :], (HH, W, DIMQ // 2)),
        jnp.broadcast_to(E[None, :, :], (HH, W, DIMQ // 2)),
    ], axis=-1)
    sp_hw = jnp.concatenate(
        [jnp.zeros((HH, W, 3 * DIMQ), jnp.float32), S4], axis=-1)

    out_ref[0] = (tok_ref[0] + add_ts[None, :, :, None, :]
                  + sp_hw[:, None, None, :, :])


def kernel(modality_tokens, timestamps, channel_embed, patch_size, input_res):
    b, h, w, t, b_s, d = modality_tokens.shape
    tok = jnp.transpose(modality_tokens, (0, 1, 3, 4, 2, 5))
    months_flat = timestamps[:, :, 1].reshape(b * t)
    gsd = (jnp.asarray(input_res).astype(jnp.float32)
           * jnp.asarray(patch_size).astype(jnp.float32)
           / float(BASE_GSD)).reshape(1, 1)
    mtab128 = jnp.concatenate(
        [_month_table(), jnp.zeros((12, DIMQ), jnp.float32)], axis=-1)
    pe = _pos_embed()
    me = _month_gather_sc(mtab128, months_flat)[:, :DIMQ].reshape(b, t, DIMQ)

    out = pl.pallas_call(
        _embed_add_kernel,
        grid=(b, h // HH),
        in_specs=[
            pl.BlockSpec((1, HH, t, b_s, w, d),
                         lambda i, j: (i, j, 0, 0, 0, 0)),
            pl.BlockSpec((1, t, DIMQ), lambda i, j: (i, 0, 0)),
            pl.BlockSpec((t, DIMQ), lambda i, j: (0, 0)),
            pl.BlockSpec((b_s, DIMQ), lambda i, j: (0, 0)),
            pl.BlockSpec((1, 1), lambda i, j: (0, 0)),
        ],
        out_specs=pl.BlockSpec((1, HH, t, b_s, w, d),
                               lambda i, j: (i, j, 0, 0, 0, 0)),
        out_shape=jax.ShapeDtypeStruct((b, h, t, b_s, w, d), jnp.float32),
        compiler_params=pltpu.CompilerParams(
            dimension_semantics=("parallel", "parallel")),
    )(tok, me, pe, channel_embed, gsd)
    return jnp.transpose(out, (0, 1, 4, 2, 3, 5))


# speedup vs baseline: 1.7389x; 1.7389x over previous
import functools

import jax
import jax.numpy as jnp
from jax import lax
from jax.experimental import pallas as pl
from jax.experimental.pallas import tpu as pltpu

BASE_GSD = 10
EMBED = 256
DIMQ = EMBED // 4
B, H, W, T, BS = 4, 16, 16, 12, 3
HH = 8


def _month_table():
    months = jnp.arange(12, dtype=jnp.float32)
    angles = 2.0 * jnp.pi * months / 12.0
    half = DIMQ // 2
    freq = jnp.arange(1, half + 1, dtype=jnp.float32)
    arg = angles[:, None] * freq[None, :]
    return jnp.concatenate([jnp.sin(arg), jnp.cos(arg)], axis=-1)


def _pos_embed():
    half = DIMQ // 2
    omega = 1.0 / (10000.0 ** (jnp.arange(half, dtype=jnp.float32) / half))
    out = jnp.arange(T, dtype=jnp.float32)[:, None] * omega
    return jnp.concatenate([jnp.sin(out), jnp.cos(out)], axis=-1)


def _embed_add_kernel(tok_ref, months_ref, mtab_ref, pe_ref, ch_ref, gsd_ref,
                      out_ref):
    j = pl.program_id(1)
    months = months_ref[0]
    mtab = mtab_ref[...]
    pe = pe_ref[...]
    ch = ch_ref[...]
    gsd = gsd_ref[0, 0]

    oh = (months == lax.broadcasted_iota(jnp.int32, (T, 12), 1))
    me = jnp.dot(oh.astype(jnp.float32), mtab,
                 preferred_element_type=jnp.float32)

    add_ts = jnp.concatenate([
        jnp.broadcast_to(ch[None, :, :], (T, BS, DIMQ)),
        jnp.broadcast_to(pe[:, None, :], (T, BS, DIMQ)),
        jnp.broadcast_to(me[:, None, :], (T, BS, DIMQ)),
        jnp.zeros((T, BS, DIMQ), jnp.float32),
    ], axis=-1)

    half2 = DIMQ // 4
    kf = lax.broadcasted_iota(jnp.int32, (1, half2), 1).astype(jnp.float32)
    omega = jnp.exp(kf * (-jnp.log(10000.0) / half2))
    posf = lax.broadcasted_iota(jnp.int32, (W, 1), 0).astype(jnp.float32) * gsd
    arg = posf * omega
    E = jnp.concatenate([jnp.sin(arg), jnp.cos(arg)], axis=-1)
    hrow = (j * HH + lax.broadcasted_iota(jnp.int32, (HH, 1), 0))
    argh = hrow.astype(jnp.float32) * gsd * omega
    Eh = jnp.concatenate([jnp.sin(argh), jnp.cos(argh)], axis=-1)
    S4 = jnp.concatenate([
        jnp.broadcast_to(Eh[:, None, :], (HH, W, DIMQ // 2)),
        jnp.broadcast_to(E[None, :, :], (HH, W, DIMQ // 2)),
    ], axis=-1)
    sp_hw = jnp.concatenate(
        [jnp.zeros((HH, W, 3 * DIMQ), jnp.float32), S4], axis=-1)

    out_ref[0] = (tok_ref[0] + add_ts[None, :, :, None, :]
                  + sp_hw[:, None, None, :, :])


def kernel(modality_tokens, timestamps, channel_embed, patch_size, input_res):
    b, h, w, t, b_s, d = modality_tokens.shape
    tok = jnp.transpose(modality_tokens, (0, 1, 3, 4, 2, 5))
    months = timestamps[:, :, 1].reshape(b, t, 1)
    gsd = (jnp.asarray(input_res).astype(jnp.float32)
           * jnp.asarray(patch_size).astype(jnp.float32)
           / float(BASE_GSD)).reshape(1, 1)
    mtab = _month_table()
    pe = _pos_embed()

    out = pl.pallas_call(
        _embed_add_kernel,
        grid=(b, h // HH),
        in_specs=[
            pl.BlockSpec((1, HH, t, b_s, w, d),
                         lambda i, j: (i, j, 0, 0, 0, 0)),
            pl.BlockSpec((1, t, 1), lambda i, j: (i, 0, 0)),
            pl.BlockSpec((12, DIMQ), lambda i, j: (0, 0)),
            pl.BlockSpec((t, DIMQ), lambda i, j: (0, 0)),
            pl.BlockSpec((b_s, DIMQ), lambda i, j: (0, 0)),
            pl.BlockSpec((1, 1), lambda i, j: (0, 0)),
        ],
        out_specs=pl.BlockSpec((1, HH, t, b_s, w, d),
                               lambda i, j: (i, j, 0, 0, 0, 0)),
        out_shape=jax.ShapeDtypeStruct((b, h, t, b_s, w, d), jnp.float32),
        compiler_params=pltpu.CompilerParams(
            dimension_semantics=("parallel", "parallel")),
    )(tok, months, mtab, pe, channel_embed, gsd)
    return jnp.transpose(out, (0, 1, 4, 2, 3, 5))


# speedup vs baseline: 1.8404x; 1.0583x over previous
import functools

import jax
import jax.numpy as jnp
from jax import lax
from jax.experimental import pallas as pl
from jax.experimental.pallas import tpu as pltpu

BASE_GSD = 10
EMBED = 256
DIMQ = EMBED // 4
B, H, W, T, BS = 4, 16, 16, 12, 3
HH = 16


def _month_table():
    months = jnp.arange(12, dtype=jnp.float32)
    angles = 2.0 * jnp.pi * months / 12.0
    half = DIMQ // 2
    freq = jnp.arange(1, half + 1, dtype=jnp.float32)
    arg = angles[:, None] * freq[None, :]
    return jnp.concatenate([jnp.sin(arg), jnp.cos(arg)], axis=-1)


def _pos_embed():
    half = DIMQ // 2
    omega = 1.0 / (10000.0 ** (jnp.arange(half, dtype=jnp.float32) / half))
    out = jnp.arange(T, dtype=jnp.float32)[:, None] * omega
    return jnp.concatenate([jnp.sin(out), jnp.cos(out)], axis=-1)


def _embed_add_kernel(tok_ref, months_ref, mtab_ref, pe_ref, ch_ref, gsd_ref,
                      out_ref):
    j = pl.program_id(1)
    months = months_ref[0]
    mtab = mtab_ref[...]
    pe = pe_ref[...]
    ch = ch_ref[...]
    gsd = gsd_ref[0, 0]

    oh = (months == lax.broadcasted_iota(jnp.int32, (T, 12), 1))
    me = jnp.dot(oh.astype(jnp.float32), mtab,
                 preferred_element_type=jnp.float32)

    add_ts = jnp.concatenate([
        jnp.broadcast_to(ch[None, :, :], (T, BS, DIMQ)),
        jnp.broadcast_to(pe[:, None, :], (T, BS, DIMQ)),
        jnp.broadcast_to(me[:, None, :], (T, BS, DIMQ)),
        jnp.zeros((T, BS, DIMQ), jnp.float32),
    ], axis=-1)

    half2 = DIMQ // 4
    kf = lax.broadcasted_iota(jnp.int32, (1, half2), 1).astype(jnp.float32)
    omega = jnp.exp(kf * (-jnp.log(10000.0) / half2))
    posf = lax.broadcasted_iota(jnp.int32, (W, 1), 0).astype(jnp.float32) * gsd
    arg = posf * omega
    E = jnp.concatenate([jnp.sin(arg), jnp.cos(arg)], axis=-1)
    hrow = (j * HH + lax.broadcasted_iota(jnp.int32, (HH, 1), 0))
    argh = hrow.astype(jnp.float32) * gsd * omega
    Eh = jnp.concatenate([jnp.sin(argh), jnp.cos(argh)], axis=-1)
    S4 = jnp.concatenate([
        jnp.broadcast_to(Eh[:, None, :], (HH, W, DIMQ // 2)),
        jnp.broadcast_to(E[None, :, :], (HH, W, DIMQ // 2)),
    ], axis=-1)
    sp_hw = jnp.concatenate(
        [jnp.zeros((HH, W, 3 * DIMQ), jnp.float32), S4], axis=-1)

    out_ref[0] = (tok_ref[0] + add_ts[None, :, :, None, :]
                  + sp_hw[:, None, None, :, :])


def kernel(modality_tokens, timestamps, channel_embed, patch_size, input_res):
    b, h, w, t, b_s, d = modality_tokens.shape
    tok = jnp.transpose(modality_tokens, (0, 1, 3, 4, 2, 5))
    months = timestamps[:, :, 1].reshape(b, t, 1)
    gsd = (jnp.asarray(input_res).astype(jnp.float32)
           * jnp.asarray(patch_size).astype(jnp.float32)
           / float(BASE_GSD)).reshape(1, 1)
    mtab = _month_table()
    pe = _pos_embed()

    out = pl.pallas_call(
        _embed_add_kernel,
        grid=(b, h // HH),
        in_specs=[
            pl.BlockSpec((1, HH, t, b_s, w, d),
                         lambda i, j: (i, j, 0, 0, 0, 0)),
            pl.BlockSpec((1, t, 1), lambda i, j: (i, 0, 0)),
            pl.BlockSpec((12, DIMQ), lambda i, j: (0, 0)),
            pl.BlockSpec((t, DIMQ), lambda i, j: (0, 0)),
            pl.BlockSpec((b_s, DIMQ), lambda i, j: (0, 0)),
            pl.BlockSpec((1, 1), lambda i, j: (0, 0)),
        ],
        out_specs=pl.BlockSpec((1, HH, t, b_s, w, d),
                               lambda i, j: (i, j, 0, 0, 0, 0)),
        out_shape=jax.ShapeDtypeStruct((b, h, t, b_s, w, d), jnp.float32),
        compiler_params=pltpu.CompilerParams(
            dimension_semantics=("parallel", "parallel")),
    )(tok, months, mtab, pe, channel_embed, gsd)
    return jnp.transpose(out, (0, 1, 4, 2, 3, 5))
